# baseline (speedup 1.0000x reference)
import jax
import jax.numpy as jnp
from jax import lax
from jax.experimental import pallas as pl
from jax.experimental.pallas import tpu as pltpu
from jax.experimental.pallas import tpu_sc as plsc

VOCAB = 100000
DIM = 512
SEQ = 200
BATCH = 1024

NC = 2
NS = 16
NW = NC * NS

N = SEQ * BATCH
PER_W = N // NW

CKD = 48
JD = 60
S = CKD * JD
CKS = 80
JS = (PER_W - S) // CKS
assert S + JS * CKS == PER_W


def _add_body(a_ref, b_ref, o_ref):
    o_ref[...] = a_ref[...] + b_ref[...]


def _sum_tables(W1, W2):
    BV = 2000
    return pl.pallas_call(
        _add_body,
        grid=(VOCAB // BV,),
        in_specs=[
            pl.BlockSpec((BV, DIM), lambda i: (i, 0)),
            pl.BlockSpec((BV, DIM), lambda i: (i, 0)),
        ],
        out_specs=pl.BlockSpec((BV, DIM), lambda i: (i, 0)),
        out_shape=jax.ShapeDtypeStruct((VOCAB, DIM), jnp.float32),
    )(W1, W2)


def _mesh():
    return plsc.VectorSubcoreMesh(
        core_axis_name="c", subcore_axis_name="s", num_cores=NC, num_subcores=NS
    )


def _direct_body(w1_hbm, w2_hbm, idx_hbm, out_hbm, idx_v, bufa0, bufa1,
                 bufb0, bufb1, asem0, asem1, bsem0, bsem1, wsem0, wsem1):
    c = lax.axis_index("c")
    s = lax.axis_index("s")
    wid = s * NC + c

    pltpu.sync_copy(idx_hbm.at[wid], idx_v)

    bufas = (bufa0, bufa1)
    bufbs = (bufb0, bufb1)
    asems = (asem0, asem1)
    bsems = (bsem0, bsem1)
    wsems = (wsem0, wsem1)

    def start_gathers(j, b):
        pltpu.async_copy(w1_hbm.at[idx_v.at[j]], bufas[b], asems[b])
        pltpu.async_copy(w2_hbm.at[idx_v.at[j]], bufbs[b], bsems[b])

    def wait_gathers(j, b):
        pltpu.make_async_copy(w1_hbm.at[idx_v.at[j]], bufas[b], asems[b]).wait()
        pltpu.make_async_copy(w2_hbm.at[idx_v.at[j]], bufbs[b], bsems[b]).wait()

    def accumulate(b):
        bufa, bufb = bufas[b], bufbs[b]

        def row(r, carry):
            for col in range(DIM // 16):
                sl = pl.ds(col * 16, 16)
                plsc.addupdate(bufa.at[r, sl], bufb[r, sl])
            return carry

        lax.fori_loop(0, CKD, row, 0)

    def start_write(j, b):
        pltpu.async_copy(bufas[b], out_hbm.at[wid, pl.ds(j * CKD, CKD)],
                         wsems[b])

    def wait_write(j, b):
        pltpu.make_async_copy(bufas[b], out_hbm.at[wid, pl.ds(j * CKD, CKD)],
                              wsems[b]).wait()

    start_gathers(0, 0)
    start_gathers(1, 1)
    wait_gathers(0, 0)
    accumulate(0)
    start_write(0, 0)

    def steady(jj, carry):
        for b, j in ((1, 2 * jj + 1), (0, 2 * jj + 2)):
            wait_write(j - 1, 1 - b)
            start_gathers(j + 1, 1 - b)
            wait_gathers(j, b)
            accumulate(b)
            start_write(j, b)
        return carry

    lax.fori_loop(0, (JD - 2) // 2, steady, 0)

    wait_write(JD - 2, 0)
    wait_gathers(JD - 1, 1)
    accumulate(1)
    start_write(JD - 1, 1)
    wait_write(JD - 1, 1)


def _sc_direct(W1, W2, idx_d):
    f = pl.kernel(
        _direct_body,
        out_type=jax.ShapeDtypeStruct((NW, PER_W, DIM), jnp.float32),
        mesh=_mesh(),
        scratch_types=[
            pltpu.VMEM((JD, CKD), jnp.int32),
            pltpu.VMEM((CKD, DIM), jnp.float32),
            pltpu.VMEM((CKD, DIM), jnp.float32),
            pltpu.VMEM((CKD, DIM), jnp.float32),
            pltpu.VMEM((CKD, DIM), jnp.float32),
            pltpu.SemaphoreType.DMA,
            pltpu.SemaphoreType.DMA,
            pltpu.SemaphoreType.DMA,
            pltpu.SemaphoreType.DMA,
            pltpu.SemaphoreType.DMA,
            pltpu.SemaphoreType.DMA,
        ],
    )
    return f(W1, W2, idx_d)


def _summed_body(tbl_hbm, idx_hbm, out_hbm, idx_v, buf0, buf1, gsem0, gsem1,
                 wsem0, wsem1):
    c = lax.axis_index("c")
    s = lax.axis_index("s")
    wid = s * NC + c

    pltpu.sync_copy(idx_hbm.at[wid], idx_v)

    bufs = (buf0, buf1)
    gsems = (gsem0, gsem1)
    wsems = (wsem0, wsem1)

    def start_gather(j, b):
        pltpu.async_copy(tbl_hbm.at[idx_v.at[j]], bufs[b], gsems[b])

    def wait_gather(j, b):
        pltpu.make_async_copy(tbl_hbm.at[idx_v.at[j]], bufs[b], gsems[b]).wait()

    def start_write(j, b):
        pltpu.async_copy(bufs[b], out_hbm.at[wid, pl.ds(S + j * CKS, CKS)],
                         wsems[b])

    def wait_write(j, b):
        pltpu.make_async_copy(bufs[b], out_hbm.at[wid, pl.ds(S + j * CKS, CKS)],
                              wsems[b]).wait()

    start_gather(0, 0)
    start_gather(1, 1)
    wait_gather(0, 0)
    start_write(0, 0)

    def steady(jj, carry):
        for b, j in ((1, 2 * jj + 1), (0, 2 * jj + 2)):
            wait_write(j - 1, 1 - b)
            start_gather(j + 1, 1 - b)
            wait_gather(j, b)
            start_write(j, b)
        return carry

    lax.fori_loop(0, (JS - 2) // 2, steady, 0)

    wait_write(JS - 2, 0)
    wait_gather(JS - 1, 1)
    start_write(JS - 1, 1)
    wait_write(JS - 1, 1)


def _sc_summed(tbl, idx_s, y_ref):
    f = pl.kernel(
        _summed_body,
        out_type=(),
        mesh=_mesh(),
        scratch_types=[
            pltpu.VMEM((JS, CKS), jnp.int32),
            pltpu.VMEM((CKS, DIM), jnp.float32),
            pltpu.VMEM((CKS, DIM), jnp.float32),
            pltpu.SemaphoreType.DMA,
            pltpu.SemaphoreType.DMA,
            pltpu.SemaphoreType.DMA,
            pltpu.SemaphoreType.DMA,
        ],
    )
    f(tbl, idx_s, y_ref)


def kernel(x, W1, W2):
    idx = x.astype(jnp.int32).reshape(NW, PER_W)
    idx_d = idx[:, :S].reshape(NW, JD, CKD)
    idx_s = idx[:, S:].reshape(NW, JS, CKS)

    y1 = _sc_direct(W1, W2, idx_d)
    tbl = _sum_tables(W1, W2)
    y_ref = jax.new_ref(y1)
    _sc_summed(tbl, idx_s, y_ref)
    return y_ref[...].reshape(SEQ, BATCH, DIM)


# speedup vs baseline: 1.0065x; 1.0065x over previous
import jax
import jax.numpy as jnp
from jax import lax
from jax.experimental import pallas as pl
from jax.experimental.pallas import tpu as pltpu
from jax.experimental.pallas import tpu_sc as plsc

VOCAB = 100000
DIM = 512
SEQ = 200
BATCH = 1024

NC = 2
NS = 16
NW = NC * NS

N = SEQ * BATCH
PER_W = N // NW

CKD = 48
JD = 60
S = CKD * JD
CKS = 80
JS = (PER_W - S) // CKS
assert S + JS * CKS == PER_W


def _add_body(a_ref, b_ref, o_ref):
    o_ref[...] = a_ref[...] + b_ref[...]


def _sum_tables(W1, W2):
    BV = 2000
    return pl.pallas_call(
        _add_body,
        grid=(VOCAB // BV,),
        in_specs=[
            pl.BlockSpec((BV, DIM), lambda i: (i, 0)),
            pl.BlockSpec((BV, DIM), lambda i: (i, 0)),
        ],
        out_specs=pl.BlockSpec((BV, DIM), lambda i: (i, 0)),
        out_shape=jax.ShapeDtypeStruct((VOCAB, DIM), jnp.float32),
    )(W1, W2)


def _mesh():
    return plsc.VectorSubcoreMesh(
        core_axis_name="c", subcore_axis_name="s", num_cores=NC, num_subcores=NS
    )


def _direct_body(w1_hbm, w2_hbm, idx_hbm, out_hbm, idx_v, bufa0, bufa1,
                 bufb0, bufb1, asem0, asem1, bsem0, bsem1, wsem0, wsem1):
    c = lax.axis_index("c")
    s = lax.axis_index("s")
    wid = s * NC + c

    pltpu.sync_copy(idx_hbm.at[wid], idx_v)

    bufas = (bufa0, bufa1)
    bufbs = (bufb0, bufb1)
    asems = (asem0, asem1)
    bsems = (bsem0, bsem1)
    wsems = (wsem0, wsem1)

    def start_gathers(j, b):
        pltpu.async_copy(w1_hbm.at[idx_v.at[j]], bufas[b], asems[b])
        pltpu.async_copy(w2_hbm.at[idx_v.at[j]], bufbs[b], bsems[b])

    def wait_gathers(j, b):
        pltpu.make_async_copy(w1_hbm.at[idx_v.at[j]], bufas[b], asems[b]).wait()
        pltpu.make_async_copy(w2_hbm.at[idx_v.at[j]], bufbs[b], bsems[b]).wait()

    def accumulate(b):
        bufa, bufb = bufas[b], bufbs[b]

        def row(r, carry):
            for col in range(DIM // 16):
                sl = pl.ds(col * 16, 16)
                bufa[r, sl] = bufa[r, sl] + bufb[r, sl]
            return carry

        lax.fori_loop(0, CKD, row, 0)

    def start_write(j, b):
        pltpu.async_copy(bufas[b], out_hbm.at[wid, pl.ds(j * CKD, CKD)],
                         wsems[b])

    def wait_write(j, b):
        pltpu.make_async_copy(bufas[b], out_hbm.at[wid, pl.ds(j * CKD, CKD)],
                              wsems[b]).wait()

    start_gathers(0, 0)
    start_gathers(1, 1)
    wait_gathers(0, 0)
    accumulate(0)
    start_write(0, 0)

    def steady(jj, carry):
        for b, j in ((1, 2 * jj + 1), (0, 2 * jj + 2)):
            wait_write(j - 1, 1 - b)
            start_gathers(j + 1, 1 - b)
            wait_gathers(j, b)
            accumulate(b)
            start_write(j, b)
        return carry

    lax.fori_loop(0, (JD - 2) // 2, steady, 0)

    wait_write(JD - 2, 0)
    wait_gathers(JD - 1, 1)
    accumulate(1)
    start_write(JD - 1, 1)
    wait_write(JD - 1, 1)


def _sc_direct(W1, W2, idx_d):
    f = pl.kernel(
        _direct_body,
        out_type=jax.ShapeDtypeStruct((NW, PER_W, DIM), jnp.float32),
        mesh=_mesh(),
        scratch_types=[
            pltpu.VMEM((JD, CKD), jnp.int32),
            pltpu.VMEM((CKD, DIM), jnp.float32),
            pltpu.VMEM((CKD, DIM), jnp.float32),
            pltpu.VMEM((CKD, DIM), jnp.float32),
            pltpu.VMEM((CKD, DIM), jnp.float32),
            pltpu.SemaphoreType.DMA,
            pltpu.SemaphoreType.DMA,
            pltpu.SemaphoreType.DMA,
            pltpu.SemaphoreType.DMA,
            pltpu.SemaphoreType.DMA,
            pltpu.SemaphoreType.DMA,
        ],
    )
    return f(W1, W2, idx_d)


def _summed_body(tbl_hbm, idx_hbm, out_hbm, idx_v, buf0, buf1, gsem0, gsem1,
                 wsem0, wsem1):
    c = lax.axis_index("c")
    s = lax.axis_index("s")
    wid = s * NC + c

    pltpu.sync_copy(idx_hbm.at[wid], idx_v)

    bufs = (buf0, buf1)
    gsems = (gsem0, gsem1)
    wsems = (wsem0, wsem1)

    def start_gather(j, b):
        pltpu.async_copy(tbl_hbm.at[idx_v.at[j]], bufs[b], gsems[b])

    def wait_gather(j, b):
        pltpu.make_async_copy(tbl_hbm.at[idx_v.at[j]], bufs[b], gsems[b]).wait()

    def start_write(j, b):
        pltpu.async_copy(bufs[b], out_hbm.at[wid, pl.ds(S + j * CKS, CKS)],
                         wsems[b])

    def wait_write(j, b):
        pltpu.make_async_copy(bufs[b], out_hbm.at[wid, pl.ds(S + j * CKS, CKS)],
                              wsems[b]).wait()

    start_gather(0, 0)
    start_gather(1, 1)
    wait_gather(0, 0)
    start_write(0, 0)

    def steady(jj, carry):
        for b, j in ((1, 2 * jj + 1), (0, 2 * jj + 2)):
            wait_write(j - 1, 1 - b)
            start_gather(j + 1, 1 - b)
            wait_gather(j, b)
            start_write(j, b)
        return carry

    lax.fori_loop(0, (JS - 2) // 2, steady, 0)

    wait_write(JS - 2, 0)
    wait_gather(JS - 1, 1)
    start_write(JS - 1, 1)
    wait_write(JS - 1, 1)


def _sc_summed(tbl, idx_s, y_ref):
    f = pl.kernel(
        _summed_body,
        out_type=(),
        mesh=_mesh(),
        scratch_types=[
            pltpu.VMEM((JS, CKS), jnp.int32),
            pltpu.VMEM((CKS, DIM), jnp.float32),
            pltpu.VMEM((CKS, DIM), jnp.float32),
            pltpu.SemaphoreType.DMA,
            pltpu.SemaphoreType.DMA,
            pltpu.SemaphoreType.DMA,
            pltpu.SemaphoreType.DMA,
        ],
    )
    f(tbl, idx_s, y_ref)


def kernel(x, W1, W2):
    idx = x.astype(jnp.int32).reshape(NW, PER_W)
    idx_d = idx[:, :S].reshape(NW, JD, CKD)
    idx_s = idx[:, S:].reshape(NW, JS, CKS)

    y1 = _sc_direct(W1, W2, idx_d)
    tbl = _sum_tables(W1, W2)
    y_ref = jax.new_ref(y1)
    _sc_summed(tbl, idx_s, y_ref)
    return y_ref[...].reshape(SEQ, BATCH, DIM)


# speedup vs baseline: 1.0691x; 1.0622x over previous
import jax
import jax.numpy as jnp
from jax import lax
from jax.experimental import pallas as pl
from jax.experimental.pallas import tpu as pltpu
from jax.experimental.pallas import tpu_sc as plsc

VOCAB = 100000
DIM = 512
SEQ = 200
BATCH = 1024

NC = 2
NS = 16
NW = NC * NS

N = SEQ * BATCH
PER_W = N // NW

CKD = 48
JD = 32
S = CKD * JD
CKS = 64
JS = (PER_W - S) // CKS
assert S + JS * CKS == PER_W


def _add_body(a_ref, b_ref, o_ref):
    o_ref[...] = a_ref[...] + b_ref[...]


def _sum_tables(W1, W2):
    BV = 2000
    return pl.pallas_call(
        _add_body,
        grid=(VOCAB // BV,),
        in_specs=[
            pl.BlockSpec((BV, DIM), lambda i: (i, 0)),
            pl.BlockSpec((BV, DIM), lambda i: (i, 0)),
        ],
        out_specs=pl.BlockSpec((BV, DIM), lambda i: (i, 0)),
        out_shape=jax.ShapeDtypeStruct((VOCAB, DIM), jnp.float32),
    )(W1, W2)


def _mesh():
    return plsc.VectorSubcoreMesh(
        core_axis_name="c", subcore_axis_name="s", num_cores=NC, num_subcores=NS
    )


def _direct_body(w1_hbm, w2_hbm, idx_hbm, out_hbm, idx_v, bufa0, bufa1,
                 bufb0, bufb1, asem0, asem1, bsem0, bsem1, wsem0, wsem1):
    c = lax.axis_index("c")
    s = lax.axis_index("s")
    wid = s * NC + c

    pltpu.sync_copy(idx_hbm.at[wid], idx_v)

    bufas = (bufa0, bufa1)
    bufbs = (bufb0, bufb1)
    asems = (asem0, asem1)
    bsems = (bsem0, bsem1)
    wsems = (wsem0, wsem1)

    def start_gathers(j, b):
        pltpu.async_copy(w1_hbm.at[idx_v.at[j]], bufas[b], asems[b])
        pltpu.async_copy(w2_hbm.at[idx_v.at[j]], bufbs[b], bsems[b])

    def wait_gathers(j, b):
        pltpu.make_async_copy(w1_hbm.at[idx_v.at[j]], bufas[b], asems[b]).wait()
        pltpu.make_async_copy(w2_hbm.at[idx_v.at[j]], bufbs[b], bsems[b]).wait()

    def accumulate(b):
        bufa, bufb = bufas[b], bufbs[b]

        def row(r, carry):
            for col in range(DIM // 16):
                sl = pl.ds(col * 16, 16)
                bufa[r, sl] = bufa[r, sl] + bufb[r, sl]
            return carry

        lax.fori_loop(0, CKD, row, 0)

    def start_write(j, b):
        pltpu.async_copy(bufas[b], out_hbm.at[wid, pl.ds(j * CKD, CKD)],
                         wsems[b])

    def wait_write(j, b):
        pltpu.make_async_copy(bufas[b], out_hbm.at[wid, pl.ds(j * CKD, CKD)],
                              wsems[b]).wait()

    start_gathers(0, 0)
    start_gathers(1, 1)
    wait_gathers(0, 0)
    accumulate(0)
    start_write(0, 0)

    def steady(jj, carry):
        for b, j in ((1, 2 * jj + 1), (0, 2 * jj + 2)):
            wait_write(j - 1, 1 - b)
            start_gathers(j + 1, 1 - b)
            wait_gathers(j, b)
            accumulate(b)
            start_write(j, b)
        return carry

    lax.fori_loop(0, (JD - 2) // 2, steady, 0)

    wait_write(JD - 2, 0)
    wait_gathers(JD - 1, 1)
    accumulate(1)
    start_write(JD - 1, 1)
    wait_write(JD - 1, 1)


def _sc_direct(W1, W2, idx_d):
    f = pl.kernel(
        _direct_body,
        out_type=jax.ShapeDtypeStruct((NW, PER_W, DIM), jnp.float32),
        mesh=_mesh(),
        scratch_types=[
            pltpu.VMEM((JD, CKD), jnp.int32),
            pltpu.VMEM((CKD, DIM), jnp.float32),
            pltpu.VMEM((CKD, DIM), jnp.float32),
            pltpu.VMEM((CKD, DIM), jnp.float32),
            pltpu.VMEM((CKD, DIM), jnp.float32),
            pltpu.SemaphoreType.DMA,
            pltpu.SemaphoreType.DMA,
            pltpu.SemaphoreType.DMA,
            pltpu.SemaphoreType.DMA,
            pltpu.SemaphoreType.DMA,
            pltpu.SemaphoreType.DMA,
        ],
    )
    return f(W1, W2, idx_d)


def _summed_body(tbl_hbm, idx_hbm, out_hbm, idx_v, buf0, buf1, gsem0, gsem1,
                 wsem0, wsem1):
    c = lax.axis_index("c")
    s = lax.axis_index("s")
    wid = s * NC + c

    pltpu.sync_copy(idx_hbm.at[wid], idx_v)

    bufs = (buf0, buf1)
    gsems = (gsem0, gsem1)
    wsems = (wsem0, wsem1)

    def start_gather(j, b):
        pltpu.async_copy(tbl_hbm.at[idx_v.at[j]], bufs[b], gsems[b])

    def wait_gather(j, b):
        pltpu.make_async_copy(tbl_hbm.at[idx_v.at[j]], bufs[b], gsems[b]).wait()

    def start_write(j, b):
        pltpu.async_copy(bufs[b], out_hbm.at[wid, pl.ds(S + j * CKS, CKS)],
                         wsems[b])

    def wait_write(j, b):
        pltpu.make_async_copy(bufs[b], out_hbm.at[wid, pl.ds(S + j * CKS, CKS)],
                              wsems[b]).wait()

    start_gather(0, 0)
    start_gather(1, 1)
    wait_gather(0, 0)
    start_write(0, 0)

    def steady(jj, carry):
        for b, j in ((1, 2 * jj + 1), (0, 2 * jj + 2)):
            wait_write(j - 1, 1 - b)
            start_gather(j + 1, 1 - b)
            wait_gather(j, b)
            start_write(j, b)
        return carry

    lax.fori_loop(0, (JS - 2) // 2, steady, 0)

    wait_write(JS - 2, 0)
    wait_gather(JS - 1, 1)
    start_write(JS - 1, 1)
    wait_write(JS - 1, 1)


def _sc_summed(tbl, idx_s, y_ref):
    f = pl.kernel(
        _summed_body,
        out_type=(),
        mesh=_mesh(),
        scratch_types=[
            pltpu.VMEM((JS, CKS), jnp.int32),
            pltpu.VMEM((CKS, DIM), jnp.float32),
            pltpu.VMEM((CKS, DIM), jnp.float32),
            pltpu.SemaphoreType.DMA,
            pltpu.SemaphoreType.DMA,
            pltpu.SemaphoreType.DMA,
            pltpu.SemaphoreType.DMA,
        ],
    )
    f(tbl, idx_s, y_ref)


def kernel(x, W1, W2):
    idx = x.astype(jnp.int32).reshape(NW, PER_W)
    idx_d = idx[:, :S].reshape(NW, JD, CKD)
    idx_s = idx[:, S:].reshape(NW, JS, CKS)

    y1 = _sc_direct(W1, W2, idx_d)
    tbl = _sum_tables(W1, W2)
    y_ref = jax.new_ref(y1)
    _sc_summed(tbl, idx_s, y_ref)
    return y_ref[...].reshape(SEQ, BATCH, DIM)


# speedup vs baseline: 1.2427x; 1.1623x over previous
import jax
import jax.numpy as jnp
from jax import lax
from jax.experimental import pallas as pl
from jax.experimental.pallas import tpu as pltpu
from jax.experimental.pallas import tpu_sc as plsc

VOCAB = 100000
DIM = 512
SEQ = 200
BATCH = 1024

NC = 2
NS = 16
NW = NC * NS

N = SEQ * BATCH
PER_W = N // NW

CKD = 40
JD = PER_W // CKD
assert CKD * JD == PER_W


def _mesh():
    return plsc.VectorSubcoreMesh(
        core_axis_name="c", subcore_axis_name="s", num_cores=NC, num_subcores=NS
    )


def _direct_body(w1_hbm, w2_hbm, idx_hbm, out_hbm, idx_v, bufa0, bufa1,
                 bufb0, bufb1, asem0, asem1, bsem0, bsem1, wsem0, wsem1):
    c = lax.axis_index("c")
    s = lax.axis_index("s")
    wid = s * NC + c

    pltpu.sync_copy(idx_hbm.at[wid], idx_v)

    bufas = (bufa0, bufa1)
    bufbs = (bufb0, bufb1)
    asems = (asem0, asem1)
    bsems = (bsem0, bsem1)
    wsems = (wsem0, wsem1)

    def start_gathers(j, b):
        pltpu.async_copy(w1_hbm.at[idx_v.at[j]], bufas[b], asems[b])
        pltpu.async_copy(w2_hbm.at[idx_v.at[j]], bufbs[b], bsems[b])

    def wait_gathers(j, b):
        pltpu.make_async_copy(w1_hbm.at[idx_v.at[j]], bufas[b], asems[b]).wait()
        pltpu.make_async_copy(w2_hbm.at[idx_v.at[j]], bufbs[b], bsems[b]).wait()

    def accumulate(b):
        bufa, bufb = bufas[b], bufbs[b]

        def row(r, carry):
            for col in range(DIM // 16):
                sl = pl.ds(col * 16, 16)
                bufa[r, sl] = bufa[r, sl] + bufb[r, sl]
            return carry

        lax.fori_loop(0, CKD, row, 0)

    def start_write(j, b):
        pltpu.async_copy(bufas[b], out_hbm.at[wid, pl.ds(j * CKD, CKD)],
                         wsems[b])

    def wait_write(j, b):
        pltpu.make_async_copy(bufas[b], out_hbm.at[wid, pl.ds(j * CKD, CKD)],
                              wsems[b]).wait()

    start_gathers(0, 0)
    start_gathers(1, 1)
    wait_gathers(0, 0)
    accumulate(0)
    start_write(0, 0)

    def steady(jj, carry):
        for b, j in ((1, 2 * jj + 1), (0, 2 * jj + 2)):
            wait_write(j - 1, 1 - b)
            start_gathers(j + 1, 1 - b)
            wait_gathers(j, b)
            accumulate(b)
            start_write(j, b)
        return carry

    lax.fori_loop(0, (JD - 2) // 2, steady, 0)

    wait_write(JD - 2, 0)
    wait_gathers(JD - 1, 1)
    accumulate(1)
    start_write(JD - 1, 1)
    wait_write(JD - 1, 1)


def _sc_direct(W1, W2, idx_d):
    f = pl.kernel(
        _direct_body,
        out_type=jax.ShapeDtypeStruct((NW, PER_W, DIM), jnp.float32),
        mesh=_mesh(),
        scratch_types=[
            pltpu.VMEM((JD, CKD), jnp.int32),
            pltpu.VMEM((CKD, DIM), jnp.float32),
            pltpu.VMEM((CKD, DIM), jnp.float32),
            pltpu.VMEM((CKD, DIM), jnp.float32),
            pltpu.VMEM((CKD, DIM), jnp.float32),
            pltpu.SemaphoreType.DMA,
            pltpu.SemaphoreType.DMA,
            pltpu.SemaphoreType.DMA,
            pltpu.SemaphoreType.DMA,
            pltpu.SemaphoreType.DMA,
            pltpu.SemaphoreType.DMA,
        ],
    )
    return f(W1, W2, idx_d)


def kernel(x, W1, W2):
    idx = x.astype(jnp.int32).reshape(NW, JD, CKD)
    y = _sc_direct(W1, W2, idx)
    return y.reshape(SEQ, BATCH, DIM)


# speedup vs baseline: 1.2504x; 1.0062x over previous
import jax
import jax.numpy as jnp
from jax import lax
from jax.experimental import pallas as pl
from jax.experimental.pallas import tpu as pltpu
from jax.experimental.pallas import tpu_sc as plsc

VOCAB = 100000
DIM = 512
SEQ = 200
BATCH = 1024

NC = 2
NS = 16
NW = NC * NS

N = SEQ * BATCH
PER_W = N // NW

CKD = 32
JD = PER_W // CKD
assert CKD * JD == PER_W
assert JD % 3 == 2


def _mesh():
    return plsc.VectorSubcoreMesh(
        core_axis_name="c", subcore_axis_name="s", num_cores=NC, num_subcores=NS
    )


def _direct_body(w1_hbm, w2_hbm, idx_hbm, out_hbm, idx_v, bufa0, bufa1, bufa2,
                 bufb0, bufb1, bufb2, asem0, asem1, asem2, bsem0, bsem1, bsem2,
                 wsem0, wsem1, wsem2):
    c = lax.axis_index("c")
    s = lax.axis_index("s")
    wid = s * NC + c

    pltpu.sync_copy(idx_hbm.at[wid], idx_v)

    bufas = (bufa0, bufa1, bufa2)
    bufbs = (bufb0, bufb1, bufb2)
    asems = (asem0, asem1, asem2)
    bsems = (bsem0, bsem1, bsem2)
    wsems = (wsem0, wsem1, wsem2)

    def start_gathers(j, b):
        pltpu.async_copy(w1_hbm.at[idx_v.at[j]], bufas[b], asems[b])
        pltpu.async_copy(w2_hbm.at[idx_v.at[j]], bufbs[b], bsems[b])

    def wait_gathers(j, b):
        pltpu.make_async_copy(w1_hbm.at[idx_v.at[j]], bufas[b], asems[b]).wait()
        pltpu.make_async_copy(w2_hbm.at[idx_v.at[j]], bufbs[b], bsems[b]).wait()

    def accumulate(b):
        bufa, bufb = bufas[b], bufbs[b]

        def row(r, carry):
            for col in range(DIM // 16):
                sl = pl.ds(col * 16, 16)
                bufa[r, sl] = bufa[r, sl] + bufb[r, sl]
            return carry

        lax.fori_loop(0, CKD, row, 0)

    def start_write(j, b):
        pltpu.async_copy(bufas[b], out_hbm.at[wid, pl.ds(j * CKD, CKD)],
                         wsems[b])

    def wait_write(j, b):
        pltpu.make_async_copy(bufas[b], out_hbm.at[wid, pl.ds(j * CKD, CKD)],
                              wsems[b]).wait()

    start_gathers(0, 0)
    start_gathers(1, 1)
    wait_gathers(0, 0)
    accumulate(0)
    start_write(0, 0)
    start_gathers(2, 2)
    wait_gathers(1, 1)
    accumulate(1)
    start_write(1, 1)
    wait_write(0, 0)
    start_gathers(3, 0)
    wait_gathers(2, 2)
    accumulate(2)
    start_write(2, 2)

    def steady(jj, carry):
        for b in range(3):
            j = 3 * jj + 3 + b
            bn = (b + 1) % 3
            wait_write(j - 2, bn)
            start_gathers(j + 1, bn)
            wait_gathers(j, b)
            accumulate(b)
            start_write(j, b)
        return carry

    lax.fori_loop(0, (JD - 5) // 3, steady, 0)

    j2, j1 = JD - 2, JD - 1
    wait_write(j2 - 2, (j2 - 2) % 3)
    start_gathers(j1, j1 % 3)
    wait_gathers(j2, j2 % 3)
    accumulate(j2 % 3)
    start_write(j2, j2 % 3)

    wait_write(j1 - 2, (j1 - 2) % 3)
    wait_gathers(j1, j1 % 3)
    accumulate(j1 % 3)
    start_write(j1, j1 % 3)

    wait_write(j2, j2 % 3)
    wait_write(j1, j1 % 3)


def _sc_direct(W1, W2, idx_d):
    f = pl.kernel(
        _direct_body,
        out_type=jax.ShapeDtypeStruct((NW, PER_W, DIM), jnp.float32),
        mesh=_mesh(),
        scratch_types=(
            [pltpu.VMEM((JD, CKD), jnp.int32)]
            + [pltpu.VMEM((CKD, DIM), jnp.float32)] * 6
            + [pltpu.SemaphoreType.DMA] * 9
        ),
    )
    return f(W1, W2, idx_d)


def kernel(x, W1, W2):
    idx = x.astype(jnp.int32).reshape(NW, JD, CKD)
    y = _sc_direct(W1, W2, idx)
    return y.reshape(SEQ, BATCH, DIM)
